# baseline (speedup 1.0000x reference)
import functools

import jax
import jax.numpy as jnp
from jax import lax
from jax.experimental import pallas as pl
from jax.experimental.pallas import tpu as pltpu
from jax.experimental.pallas import tpu_sc as plsc

_NUM_CORES = 2
_NUM_SUBCORES = 16
_NUM_WORKERS = _NUM_CORES * _NUM_SUBCORES


def _sc_gather(ids, weight, chunk_rows):
    b = ids.shape[0]
    _, d = weight.shape
    b_per_w = b // _NUM_WORKERS
    n_chunks = b_per_w // chunk_rows
    assert n_chunks % 2 == 0 and n_chunks >= 4
    mesh = plsc.VectorSubcoreMesh(core_axis_name="c", subcore_axis_name="s")

    @functools.partial(
        pl.kernel,
        out_type=jax.ShapeDtypeStruct((b, d), jnp.float32),
        mesh=mesh,
        scratch_types=[
            pltpu.VMEM((b_per_w,), jnp.int32),
            pltpu.VMEM((chunk_rows, d), jnp.float32),
            pltpu.VMEM((chunk_rows, d), jnp.float32),
            pltpu.SemaphoreType.DMA,
            pltpu.SemaphoreType.DMA,
            pltpu.SemaphoreType.DMA,
            pltpu.SemaphoreType.DMA,
        ],
    )
    def gather_kernel(idx_hbm, table_hbm, out_hbm, idx_v, rows0, rows1,
                      gsem0, gsem1, wsem0, wsem1):
        wid = lax.axis_index("s") * _NUM_CORES + lax.axis_index("c")
        base = wid * b_per_w
        pltpu.sync_copy(idx_hbm.at[pl.ds(base, b_per_w)], idx_v)

        bufs = (rows0, rows1)
        gsems = (gsem0, gsem1)
        wsems = (wsem0, wsem1)

        def gather_copy(c, bi):
            rows = idx_v.at[pl.ds(c * chunk_rows, chunk_rows)]
            return pltpu.make_async_copy(table_hbm.at[rows], bufs[bi], gsems[bi])

        def write_copy(c, bi):
            dst = out_hbm.at[pl.ds(base + c * chunk_rows, chunk_rows)]
            return pltpu.make_async_copy(bufs[bi], dst, wsems[bi])

        gather_copy(0, 0).start()
        gather_copy(1, 1).start()

        @pl.loop(0, n_chunks - 2, step=2)
        def _(i):
            for bi in range(2):
                c = i + bi
                gather_copy(c, bi).wait()
                write_copy(c, bi).start()
                write_copy(c, bi).wait()
                gather_copy(c + 2, bi).start()

        for bi in range(2):
            c = n_chunks - 2 + bi
            gather_copy(c, bi).wait()
            write_copy(c, bi).start()
            write_copy(c, bi).wait()

    return gather_kernel(ids, weight)


def kernel(input_ids, weight):
    if input_ids.size == 0:
        return jnp.zeros((0, weight.shape[1]), dtype=jnp.float32)
    ids = input_ids.reshape(-1).astype(jnp.int32)
    out = _sc_gather(ids, weight, chunk_rows=16)
    return out.reshape(*input_ids.shape, weight.shape[1])


# speedup vs baseline: 1.0233x; 1.0233x over previous
import functools

import jax
import jax.numpy as jnp
from jax import lax
from jax.experimental import pallas as pl
from jax.experimental.pallas import tpu as pltpu
from jax.experimental.pallas import tpu_sc as plsc

_NUM_CORES = 2
_NUM_SUBCORES = 16
_NUM_WORKERS = _NUM_CORES * _NUM_SUBCORES


def _sc_gather(ids, weight, chunk_rows):
    b = ids.shape[0]
    _, d = weight.shape
    b_per_w = b // _NUM_WORKERS
    n_chunks = b_per_w // chunk_rows
    assert n_chunks % 2 == 0 and n_chunks >= 4
    mesh = plsc.VectorSubcoreMesh(core_axis_name="c", subcore_axis_name="s")

    @functools.partial(
        pl.kernel,
        out_type=jax.ShapeDtypeStruct((b, d), jnp.float32),
        mesh=mesh,
        scratch_types=[
            pltpu.VMEM((b_per_w,), jnp.int32),
            pltpu.VMEM((chunk_rows, d), jnp.float32),
            pltpu.VMEM((chunk_rows, d), jnp.float32),
            pltpu.SemaphoreType.DMA,
            pltpu.SemaphoreType.DMA,
            pltpu.SemaphoreType.DMA,
            pltpu.SemaphoreType.DMA,
        ],
    )
    def gather_kernel(idx_hbm, table_hbm, out_hbm, idx_v, rows0, rows1,
                      gsem0, gsem1, wsem0, wsem1):
        wid = lax.axis_index("s") * _NUM_CORES + lax.axis_index("c")
        base = wid * b_per_w
        pltpu.sync_copy(idx_hbm.at[pl.ds(base, b_per_w)], idx_v)

        bufs = (rows0, rows1)
        gsems = (gsem0, gsem1)
        wsems = (wsem0, wsem1)

        def gather_copy(c, bi):
            rows = idx_v.at[pl.ds(c * chunk_rows, chunk_rows)]
            return pltpu.make_async_copy(table_hbm.at[rows], bufs[bi], gsems[bi])

        def write_copy(c, bi):
            dst = out_hbm.at[pl.ds(base + c * chunk_rows, chunk_rows)]
            return pltpu.make_async_copy(bufs[bi], dst, wsems[bi])

        gather_copy(0, 0).start()
        gather_copy(1, 1).start()

        @pl.loop(0, n_chunks - 2, step=2)
        def _(i):
            for bi in range(2):
                c = i + bi
                gather_copy(c, bi).wait()
                write_copy(c, bi).start()
                write_copy(c, bi).wait()
                gather_copy(c + 2, bi).start()

        for bi in range(2):
            c = n_chunks - 2 + bi
            gather_copy(c, bi).wait()
            write_copy(c, bi).start()
            write_copy(c, bi).wait()

    return gather_kernel(ids, weight)


def kernel(input_ids, weight):
    if input_ids.size == 0:
        return jnp.zeros((0, weight.shape[1]), dtype=jnp.float32)
    ids = input_ids.reshape(-1).astype(jnp.int32)
    out = _sc_gather(ids, weight, chunk_rows=32)
    return out.reshape(*input_ids.shape, weight.shape[1])
